# baseline (speedup 1.0000x reference)
import functools

import jax
import jax.numpy as jnp
from jax import lax
from jax.experimental import pallas as pl
from jax.experimental.pallas import tpu as pltpu
from jax.experimental.pallas import tpu_sc as plsc

_N = 10000
_E = 160000
_D = 128
_B = 16
_NPB = _N // _B
_L = 4

_NTILES = 16
_EP = _E // _NTILES
_K = 40
_STEPS = _EP // _K
_MUL_UNROLL = 8

_RPT = 624
_TAIL_OFF = _RPT * _NTILES
_TAIL = _N - _TAIL_OFF

_DW = _D // 2

_QPERM = tuple(
    32 * g + (16 if p % 2 else 0) + (p // 2)
    for g in range(_D // 32) for p in range(32))


def _partitioned_rowcopy(s, copy_fn):
    copy_fn(s * _RPT, _RPT)

    @pl.when(s == _NTILES - 1)
    def _():
        copy_fn(_TAIL_OFF, _TAIL)


def _sc_counts_body(dst_s_hbm, dst_t_hbm, zeros_hbm, out_s, out_t,
                    didx_v, ones_v, cnt_sh, sem_sc):
    c = lax.axis_index("c")
    s = lax.axis_index("s")
    one = jnp.full((16,), 1.0, dtype=jnp.float32)

    def rowinit(i, _):
        for j in range(_D // 16):
            ones_v[i, pl.ds(j * 16, 16)] = one
        return 0
    lax.fori_loop(0, _K, rowinit, 0)

    _partitioned_rowcopy(s, lambda off, sz: pltpu.sync_copy(
        zeros_hbm.at[pl.ds(off, sz)], cnt_sh.at[pl.ds(off, sz)]))

    def run(dst_hbm, out_hbm, sem):
        pltpu.sync_copy(dst_hbm.at[s], didx_v)
        plsc.subcore_barrier()

        def scat_start(k):
            pltpu.async_copy(ones_v, cnt_sh.at[didx_v.at[k]], sem, add=True)

        def scat_wait(k):
            pltpu.make_async_copy(ones_v, cnt_sh.at[didx_v.at[k]], sem).wait()

        depth = 8
        for k in range(depth):
            scat_start(k)

        def step(k, _):
            scat_wait(k)
            scat_start(k + depth)
            return 0
        lax.fori_loop(0, _STEPS - depth, step, 0)
        for k in range(depth):
            scat_wait(_STEPS - depth + k)
        plsc.subcore_barrier()
        _partitioned_rowcopy(s, lambda off, sz: pltpu.sync_copy(
            cnt_sh.at[pl.ds(off, sz)], out_hbm.at[pl.ds(off, sz)]))

    @pl.when(c == 0)
    def _():
        run(dst_s_hbm, out_s, sem_sc)

    @pl.when(c == 1)
    def _():
        run(dst_t_hbm, out_t, sem_sc)


def _sc_messages_body(xs_hbm, xt_hbm,
                      src_s_hbm, dst_s_hbm, attr_s_hbm,
                      src_t_hbm, dst_t_hbm, attr_t_hbm,
                      zeros_hbm, out_s, out_t,
                      sidx0, sidx1, sidx2, didx0, didx1, didx2,
                      attr0, attr1, attr2, xr0, xr1, xr2,
                      msg0, msg1, msg2, agg_sh,
                      sem_a0, sem_a1, sem_a2, sem_g0, sem_g1, sem_g2,
                      sem_i0, sem_i1, sem_i2, sem_s0, sem_s1, sem_s2):
    c = lax.axis_index("c")
    s = lax.axis_index("s")

    _partitioned_rowcopy(s, lambda off, sz: pltpu.sync_copy(
        zeros_hbm.at[pl.ds(off, sz)], agg_sh.at[pl.ds(off, sz)]))
    plsc.subcore_barrier()

    sidx = (sidx0, sidx1, sidx2)
    didx = (didx0, didx1, didx2)
    attrs = (attr0, attr1, attr2)
    xrs = (xr0, xr1, xr2)
    msgs = (msg0, msg1, msg2)
    sem_a = (sem_a0, sem_a1, sem_a2)
    sem_g = (sem_g0, sem_g1, sem_g2)
    sem_i = (sem_i0, sem_i1, sem_i2)
    sem_s = (sem_s0, sem_s1, sem_s2)

    def run(x_hbm, src_hbm, dst_hbm, attr_hbm, out_hbm):
        def idx_copies(k, b):
            base = s * _EP + k * _K
            return (
                pltpu.make_async_copy(src_hbm.at[pl.ds(base, _K)],
                                      sidx[b], sem_i[b]),
                pltpu.make_async_copy(dst_hbm.at[pl.ds(base, _K)],
                                      didx[b], sem_i[b]),
            )

        def attr_copy(k, b):
            base = (s * _EP + k * _K) * _DW
            return pltpu.make_async_copy(attr_hbm.at[pl.ds(base, _K * _DW)],
                                         attrs[b], sem_a[b])

        def gather_copy(k, b):
            del k
            return pltpu.make_async_copy(x_hbm.at[sidx[b]], xrs[b], sem_g[b])

        def scat_start(b):
            pltpu.async_copy(msgs[b], agg_sh.at[didx[b]], sem_s[b], add=True)

        def scat_wait(b):
            pltpu.make_async_copy(msgs[b], agg_sh.at[didx[b]],
                                  sem_s[b]).wait()

        def in_start(k, b):
            for cp in idx_copies(k, b):
                cp.start()
            attr_copy(k, b).start()

        def mul(b):
            def mrows(i, _):
                for r in range(_MUL_UNROLL):
                    row = i * _MUL_UNROLL + r
                    for g in range(_D // 32):
                        w = attrs[b][pl.ds(row * _DW + 16 * g, 16)]
                        aa = lax.bitcast_convert_type(
                            lax.shift_left(w, 16), jnp.float32)
                        ab = lax.bitcast_convert_type(
                            lax.shift_left(lax.shift_right_logical(w, 16), 16),
                            jnp.float32)
                        sl0 = pl.ds(32 * g, 16)
                        sl1 = pl.ds(32 * g + 16, 16)
                        msgs[b][row, sl0] = aa * xrs[b][row, sl0]
                        msgs[b][row, sl1] = ab * xrs[b][row, sl1]
                return 0
            lax.fori_loop(0, _K // _MUL_UNROLL, mrows, 0)

        def step(k, b, b1, b2):
            @pl.when(k + 2 < _STEPS)
            def _():
                scat_wait(b2)
                in_start(k + 2, b2)

            @pl.when(k + 1 < _STEPS)
            def _():
                for cp in idx_copies(k + 1, b1):
                    cp.wait()
                gather_copy(k + 1, b1).start()
            attr_copy(k, b).wait()
            gather_copy(k, b).wait()
            mul(b)
            scat_start(b)

        in_start(0, 0)
        in_start(1, 1)
        for cp in idx_copies(0, 0):
            cp.wait()
        gather_copy(0, 0).start()
        in_start(2, 2)
        for cp in idx_copies(1, 1):
            cp.wait()
        gather_copy(1, 1).start()
        attr_copy(0, 0).wait()
        gather_copy(0, 0).wait()
        mul(0)
        scat_start(0)

        def trip(i, _):
            k0 = 3 * i
            step(k0 + 1, 1, 2, 0)
            step(k0 + 2, 2, 0, 1)
            step(k0 + 3, 0, 1, 2)
            return 0
        lax.fori_loop(0, (_STEPS - 1) // 3, trip, 0)

        scat_wait(1)
        scat_wait(2)
        scat_wait(0)

        plsc.subcore_barrier()
        _partitioned_rowcopy(s, lambda off, sz: pltpu.sync_copy(
            agg_sh.at[pl.ds(off, sz)], out_hbm.at[pl.ds(off, sz)]))

    @pl.when(c == 0)
    def _():
        run(xs_hbm, src_s_hbm, dst_s_hbm, attr_s_hbm, out_s)

    @pl.when(c == 1)
    def _():
        run(xt_hbm, src_t_hbm, dst_t_hbm, attr_t_hbm, out_t)


@functools.lru_cache(maxsize=None)
def _build_sc_kernels():
    mesh = plsc.VectorSubcoreMesh(core_axis_name="c", subcore_axis_name="s",
                                  num_cores=2, num_subcores=_NTILES)
    out2 = (jax.ShapeDtypeStruct((_N, _D), jnp.float32),
            jax.ShapeDtypeStruct((_N, _D), jnp.float32))
    counts = pl.kernel(
        _sc_counts_body, out_type=out2, mesh=mesh,
        scratch_types=[
            pltpu.VMEM((_STEPS, _K), jnp.int32),
            pltpu.VMEM((_K, _D), jnp.float32),
            pltpu.VMEM_SHARED((_N, _D), jnp.float32),
            pltpu.SemaphoreType.DMA,
        ])
    messages = pl.kernel(
        _sc_messages_body, out_type=out2, mesh=mesh,
        scratch_types=(
            [pltpu.VMEM((_K,), jnp.int32)] * 6
            + [pltpu.VMEM((_K * _DW,), jnp.int32)] * 3
            + [pltpu.VMEM((_K, _D), jnp.float32)] * 3
            + [pltpu.VMEM((_K, _D), jnp.float32)] * 3
            + [pltpu.VMEM_SHARED((_N, _D), jnp.float32)]
            + [pltpu.SemaphoreType.DMA] * 12
        ))
    return counts, messages


def _dense_heads(aggs, aggt, cnts, cntt, xs, xt, Wr, br, Wroot):
    ms = aggs / jnp.maximum(cnts, 1.0)
    mt = aggt / jnp.maximum(cntt, 1.0)
    hs = jnp.maximum(
        jnp.dot(ms, Wr, preferred_element_type=jnp.float32) + br
        + jnp.dot(xs, Wroot, preferred_element_type=jnp.float32), 0.0)
    ht = jnp.maximum(
        jnp.dot(mt, Wr, preferred_element_type=jnp.float32) + br
        + jnp.dot(xt, Wroot, preferred_element_type=jnp.float32), 0.0)
    return hs, ht


def _attention(hs, ht):
    sim = lax.dot_general(hs, ht, (((1,), (1,)), ((), ())),
                          preferred_element_type=jnp.float32)
    es = jnp.exp(sim - jnp.max(sim, axis=1, keepdims=True))
    attn_s = es / jnp.sum(es, axis=1, keepdims=True)
    new_s = jnp.dot(attn_s, ht, preferred_element_type=jnp.float32)
    et = jnp.exp(sim - jnp.max(sim, axis=0, keepdims=True))
    attn_t = et / jnp.sum(et, axis=0, keepdims=True)
    new_t = lax.dot_general(attn_t, hs, (((0,), (0,)), ((), ())),
                            preferred_element_type=jnp.float32)
    return new_s, new_t


def _tc_layer_body(aggs_ref, aggt_ref, cnts_ref, cntt_ref, xs_ref, xt_ref,
                   Wr_ref, br_ref, Wroot_ref, os_ref, ot_ref):
    hs, ht = _dense_heads(aggs_ref[0], aggt_ref[0], cnts_ref[0], cntt_ref[0],
                          xs_ref[0], xt_ref[0], Wr_ref[...], br_ref[...],
                          Wroot_ref[...])
    new_s, new_t = _attention(hs, ht)
    os_ref[0] = new_s
    ot_ref[0] = new_t


def _tc_last_body(aggs_ref, aggt_ref, cnts_ref, cntt_ref, xs_ref, xt_ref,
                  Wr_ref, br_ref, Wroot_ref, sm_ref, tm_ref):
    hs, ht = _dense_heads(aggs_ref[0], aggt_ref[0], cnts_ref[0], cntt_ref[0],
                          xs_ref[0], xt_ref[0], Wr_ref[...], br_ref[...],
                          Wroot_ref[...])
    new_s, new_t = _attention(hs, ht)
    sm_ref[...] = jnp.mean(new_s, axis=0, keepdims=True)[None]
    tm_ref[...] = jnp.mean(new_t, axis=0, keepdims=True)[None]


def _tc_cos_body(sm_ref, tm_ref, out_ref):
    sm = sm_ref[...]
    tm = tm_ref[...]
    dot = jnp.sum(sm * tm, axis=1, keepdims=True)
    ns = jnp.sqrt(jnp.sum(sm * sm, axis=1, keepdims=True))
    nt = jnp.sqrt(jnp.sum(tm * tm, axis=1, keepdims=True))
    out_ref[...] = dot / jnp.maximum(ns * nt, 1e-8)


_blk = pl.BlockSpec((1, _NPB, _D), lambda b: (b, 0, 0))
_wblk = pl.BlockSpec((_D, _D), lambda b: (0, 0))
_bblk = pl.BlockSpec((1, _D), lambda b: (0, 0))
_mblk = pl.BlockSpec((1, 1, _D), lambda b: (b, 0, 0))

_tc_layer = pl.pallas_call(
    _tc_layer_body,
    grid=(_B,),
    in_specs=[_blk, _blk, _blk, _blk, _blk, _blk, _wblk, _bblk, _wblk],
    out_specs=[_blk, _blk],
    out_shape=(
        jax.ShapeDtypeStruct((_B, _NPB, _D), jnp.float32),
        jax.ShapeDtypeStruct((_B, _NPB, _D), jnp.float32),
    ),
)

_tc_last = pl.pallas_call(
    _tc_last_body,
    grid=(_B,),
    in_specs=[_blk, _blk, _blk, _blk, _blk, _blk, _wblk, _bblk, _wblk],
    out_specs=[_mblk, _mblk],
    out_shape=(
        jax.ShapeDtypeStruct((_B, 1, _D), jnp.float32),
        jax.ShapeDtypeStruct((_B, 1, _D), jnp.float32),
    ),
)

_tc_cos = pl.pallas_call(
    _tc_cos_body,
    out_shape=jax.ShapeDtypeStruct((_B, 1), jnp.float32),
)


def _pack_bf16(a):
    b = a[:, _QPERM].astype(jnp.bfloat16).reshape(a.shape[0], _DW, 2)
    return lax.bitcast_convert_type(b, jnp.int32)


def kernel(src_x, tgt_x, src_edge_attr, tgt_edge_attr, W_rel, b_rel, W_root,
           src_edge_index, tgt_edge_index, src_batch, tgt_batch):
    del src_batch, tgt_batch
    zeros = jnp.zeros((_N, _D), jnp.float32)
    src_s, src_d = src_edge_index[0], src_edge_index[1]
    tgt_s, tgt_d = tgt_edge_index[0], tgt_edge_index[1]
    attr_sp = _pack_bf16(src_edge_attr).reshape(-1)
    attr_tp = _pack_bf16(tgt_edge_attr).reshape(-1)
    src_d_r = src_d.reshape(_NTILES, _STEPS, _K)
    tgt_d_r = tgt_d.reshape(_NTILES, _STEPS, _K)

    _sc_counts, _sc_messages = _build_sc_kernels()
    cnt_s, cnt_t = _sc_counts(src_d_r, tgt_d_r, zeros)
    cnt_s = cnt_s.reshape(_B, _NPB, _D)
    cnt_t = cnt_t.reshape(_B, _NPB, _D)

    xs, xt = src_x, tgt_x
    for i in range(_L):
        agg_s, agg_t = _sc_messages(xs, xt, src_s, src_d, attr_sp,
                                    tgt_s, tgt_d, attr_tp, zeros)
        args = (agg_s.reshape(_B, _NPB, _D), agg_t.reshape(_B, _NPB, _D),
                cnt_s, cnt_t,
                xs.reshape(_B, _NPB, _D), xt.reshape(_B, _NPB, _D),
                W_rel[i], b_rel[i].reshape(1, _D), W_root[i])
        if i < _L - 1:
            ys, yt = _tc_layer(*args)
            xs, xt = ys.reshape(_N, _D), yt.reshape(_N, _D)
        else:
            sm, tm = _tc_last(*args)

    return _tc_cos(sm.reshape(_B, _D), tm.reshape(_B, _D)).reshape(_B)


# speedup vs baseline: 2.7296x; 2.7296x over previous
import functools

import jax
import jax.numpy as jnp
from jax import lax
from jax.experimental import pallas as pl
from jax.experimental.pallas import tpu as pltpu
from jax.experimental.pallas import tpu_sc as plsc

_N = 10000
_E = 160000
_D = 128
_B = 16
_NPB = _N // _B
_L = 4

_NTILES = 16
_EP = _E // _NTILES
_K = 40
_STEPS = _EP // _K
_MUL_UNROLL = 8

_RPT = 624
_TAIL_OFF = _RPT * _NTILES
_TAIL = _N - _TAIL_OFF


def _partitioned_rowcopy(s, copy_fn):
    copy_fn(s * _RPT, _RPT)

    @pl.when(s == _NTILES - 1)
    def _():
        copy_fn(_TAIL_OFF, _TAIL)


def _sc_counts_body(dst_s_hbm, dst_t_hbm, zeros_hbm, out_s, out_t,
                    didx_v, ones_v, cnt_sh, sem_sc):
    c = lax.axis_index("c")
    s = lax.axis_index("s")
    one = jnp.full((16,), 1.0, dtype=jnp.float32)

    def rowinit(i, _):
        for j in range(_D // 16):
            ones_v[i, pl.ds(j * 16, 16)] = one
        return 0
    lax.fori_loop(0, _K, rowinit, 0)

    _partitioned_rowcopy(s, lambda off, sz: pltpu.sync_copy(
        zeros_hbm.at[pl.ds(off, sz)], cnt_sh.at[pl.ds(off, sz)]))

    def run(dst_hbm, out_hbm, sem):
        pltpu.sync_copy(dst_hbm.at[s], didx_v)
        plsc.subcore_barrier()

        def scat_start(k):
            pltpu.async_copy(ones_v, cnt_sh.at[didx_v.at[k]], sem, add=True)

        def scat_wait(k):
            pltpu.make_async_copy(ones_v, cnt_sh.at[didx_v.at[k]], sem).wait()

        depth = 8
        for k in range(depth):
            scat_start(k)

        def step(k, _):
            scat_wait(k)
            scat_start(k + depth)
            return 0
        lax.fori_loop(0, _STEPS - depth, step, 0)
        for k in range(depth):
            scat_wait(_STEPS - depth + k)
        plsc.subcore_barrier()
        _partitioned_rowcopy(s, lambda off, sz: pltpu.sync_copy(
            cnt_sh.at[pl.ds(off, sz)], out_hbm.at[pl.ds(off, sz)]))

    @pl.when(c == 0)
    def _():
        run(dst_s_hbm, out_s, sem_sc)

    @pl.when(c == 1)
    def _():
        run(dst_t_hbm, out_t, sem_sc)


def _sc_messages_body(xs_hbm, xt_hbm,
                      src_s_hbm, dst_s_hbm, attr_s_hbm,
                      src_t_hbm, dst_t_hbm, attr_t_hbm,
                      zeros_hbm, out_s, out_t,
                      sidx0, sidx1, sidx2, didx0, didx1, didx2,
                      attr0, attr1, attr2, xr0, xr1, xr2, agg_sh,
                      sem_a0, sem_a1, sem_a2, sem_g0, sem_g1, sem_g2,
                      sem_i0, sem_i1, sem_i2, sem_s0, sem_s1, sem_s2):
    c = lax.axis_index("c")
    s = lax.axis_index("s")

    _partitioned_rowcopy(s, lambda off, sz: pltpu.sync_copy(
        zeros_hbm.at[pl.ds(off, sz)], agg_sh.at[pl.ds(off, sz)]))
    plsc.subcore_barrier()

    sidx = (sidx0, sidx1, sidx2)
    didx = (didx0, didx1, didx2)
    attrs = (attr0, attr1, attr2)
    xrs = (xr0, xr1, xr2)
    sem_a = (sem_a0, sem_a1, sem_a2)
    sem_g = (sem_g0, sem_g1, sem_g2)
    sem_i = (sem_i0, sem_i1, sem_i2)
    sem_s = (sem_s0, sem_s1, sem_s2)

    def run(x_hbm, src_hbm, dst_hbm, attr_hbm, out_hbm):
        def idx_copies(k, b):
            base = s * _EP + k * _K
            return (
                pltpu.make_async_copy(src_hbm.at[pl.ds(base, _K)],
                                      sidx[b], sem_i[b]),
                pltpu.make_async_copy(dst_hbm.at[pl.ds(base, _K)],
                                      didx[b], sem_i[b]),
            )

        def attr_copy(k, b):
            base = s * _EP + k * _K
            return pltpu.make_async_copy(attr_hbm.at[pl.ds(base, _K)],
                                         attrs[b], sem_a[b])

        def gather_copy(k, b):
            del k
            return pltpu.make_async_copy(x_hbm.at[sidx[b]], xrs[b], sem_g[b])

        def scat_start(b):
            pltpu.async_copy(attrs[b], agg_sh.at[didx[b]], sem_s[b], add=True)

        def scat_wait(b):
            pltpu.make_async_copy(attrs[b], agg_sh.at[didx[b]],
                                  sem_s[b]).wait()

        def in_start(k, b):
            for cp in idx_copies(k, b):
                cp.start()
            attr_copy(k, b).start()

        def mul(b):
            def mrows(i, _):
                for r in range(_MUL_UNROLL):
                    row = i * _MUL_UNROLL + r
                    for j in range(_D // 16):
                        sl = pl.ds(j * 16, 16)
                        attrs[b][row, sl] = attrs[b][row, sl] * xrs[b][row, sl]
                return 0
            lax.fori_loop(0, _K // _MUL_UNROLL, mrows, 0)

        def step(k, b, b1, b2):
            @pl.when(k + 2 < _STEPS)
            def _():
                scat_wait(b2)
                in_start(k + 2, b2)

            @pl.when(k + 1 < _STEPS)
            def _():
                for cp in idx_copies(k + 1, b1):
                    cp.wait()
                gather_copy(k + 1, b1).start()
            attr_copy(k, b).wait()
            gather_copy(k, b).wait()
            mul(b)
            scat_start(b)

        in_start(0, 0)
        in_start(1, 1)
        for cp in idx_copies(0, 0):
            cp.wait()
        gather_copy(0, 0).start()
        in_start(2, 2)
        for cp in idx_copies(1, 1):
            cp.wait()
        gather_copy(1, 1).start()
        attr_copy(0, 0).wait()
        gather_copy(0, 0).wait()
        mul(0)
        scat_start(0)

        def trip(i, _):
            k0 = 3 * i
            step(k0 + 1, 1, 2, 0)
            step(k0 + 2, 2, 0, 1)
            step(k0 + 3, 0, 1, 2)
            return 0
        lax.fori_loop(0, (_STEPS - 1) // 3, trip, 0)

        scat_wait(1)
        scat_wait(2)
        scat_wait(0)

        plsc.subcore_barrier()
        _partitioned_rowcopy(s, lambda off, sz: pltpu.sync_copy(
            agg_sh.at[pl.ds(off, sz)], out_hbm.at[pl.ds(off, sz)]))

    @pl.when(c == 0)
    def _():
        run(xs_hbm, src_s_hbm, dst_s_hbm, attr_s_hbm, out_s)

    @pl.when(c == 1)
    def _():
        run(xt_hbm, src_t_hbm, dst_t_hbm, attr_t_hbm, out_t)


@functools.lru_cache(maxsize=None)
def _build_sc_kernels():
    mesh = plsc.VectorSubcoreMesh(core_axis_name="c", subcore_axis_name="s",
                                  num_cores=2, num_subcores=_NTILES)
    out2 = (jax.ShapeDtypeStruct((_N, _D), jnp.float32),
            jax.ShapeDtypeStruct((_N, _D), jnp.float32))
    counts = pl.kernel(
        _sc_counts_body, out_type=out2, mesh=mesh,
        scratch_types=[
            pltpu.VMEM((_STEPS, _K), jnp.int32),
            pltpu.VMEM((_K, _D), jnp.float32),
            pltpu.VMEM_SHARED((_N, _D), jnp.float32),
            pltpu.SemaphoreType.DMA,
        ])
    messages = pl.kernel(
        _sc_messages_body, out_type=out2, mesh=mesh,
        scratch_types=(
            [pltpu.VMEM((_K,), jnp.int32)] * 6
            + [pltpu.VMEM((_K, _D), jnp.float32)] * 6
            + [pltpu.VMEM_SHARED((_N, _D), jnp.float32)]
            + [pltpu.SemaphoreType.DMA] * 12
        ))
    return counts, messages


def _dense_heads(aggs, aggt, cnts, cntt, xs, xt, Wr, br, Wroot):
    ms = aggs / jnp.maximum(cnts, 1.0)
    mt = aggt / jnp.maximum(cntt, 1.0)
    hs = jnp.maximum(
        jnp.dot(ms, Wr, preferred_element_type=jnp.float32) + br
        + jnp.dot(xs, Wroot, preferred_element_type=jnp.float32), 0.0)
    ht = jnp.maximum(
        jnp.dot(mt, Wr, preferred_element_type=jnp.float32) + br
        + jnp.dot(xt, Wroot, preferred_element_type=jnp.float32), 0.0)
    return hs, ht


def _attention(hs, ht):
    sim = lax.dot_general(hs, ht, (((1,), (1,)), ((), ())),
                          preferred_element_type=jnp.float32)
    es = jnp.exp(sim - jnp.max(sim, axis=1, keepdims=True))
    attn_s = es / jnp.sum(es, axis=1, keepdims=True)
    new_s = jnp.dot(attn_s, ht, preferred_element_type=jnp.float32)
    et = jnp.exp(sim - jnp.max(sim, axis=0, keepdims=True))
    attn_t = et / jnp.sum(et, axis=0, keepdims=True)
    new_t = lax.dot_general(attn_t, hs, (((0,), (0,)), ((), ())),
                            preferred_element_type=jnp.float32)
    return new_s, new_t


def _tc_layer_body(aggs_ref, aggt_ref, cnts_ref, cntt_ref, xs_ref, xt_ref,
                   Wr_ref, br_ref, Wroot_ref, os_ref, ot_ref):
    hs, ht = _dense_heads(aggs_ref[0], aggt_ref[0], cnts_ref[0], cntt_ref[0],
                          xs_ref[0], xt_ref[0], Wr_ref[...], br_ref[...],
                          Wroot_ref[...])
    new_s, new_t = _attention(hs, ht)
    os_ref[0] = new_s
    ot_ref[0] = new_t


def _tc_last_body(aggs_ref, aggt_ref, cnts_ref, cntt_ref, xs_ref, xt_ref,
                  Wr_ref, br_ref, Wroot_ref, sm_ref, tm_ref):
    hs, ht = _dense_heads(aggs_ref[0], aggt_ref[0], cnts_ref[0], cntt_ref[0],
                          xs_ref[0], xt_ref[0], Wr_ref[...], br_ref[...],
                          Wroot_ref[...])
    new_s, new_t = _attention(hs, ht)
    sm_ref[...] = jnp.mean(new_s, axis=0, keepdims=True)[None]
    tm_ref[...] = jnp.mean(new_t, axis=0, keepdims=True)[None]


def _tc_cos_body(sm_ref, tm_ref, out_ref):
    sm = sm_ref[...]
    tm = tm_ref[...]
    dot = jnp.sum(sm * tm, axis=1, keepdims=True)
    ns = jnp.sqrt(jnp.sum(sm * sm, axis=1, keepdims=True))
    nt = jnp.sqrt(jnp.sum(tm * tm, axis=1, keepdims=True))
    out_ref[...] = dot / jnp.maximum(ns * nt, 1e-8)


_blk = pl.BlockSpec((1, _NPB, _D), lambda b: (b, 0, 0))
_wblk = pl.BlockSpec((_D, _D), lambda b: (0, 0))
_bblk = pl.BlockSpec((1, _D), lambda b: (0, 0))
_mblk = pl.BlockSpec((1, 1, _D), lambda b: (b, 0, 0))

_tc_layer = pl.pallas_call(
    _tc_layer_body,
    grid=(_B,),
    in_specs=[_blk, _blk, _blk, _blk, _blk, _blk, _wblk, _bblk, _wblk],
    out_specs=[_blk, _blk],
    out_shape=(
        jax.ShapeDtypeStruct((_B, _NPB, _D), jnp.float32),
        jax.ShapeDtypeStruct((_B, _NPB, _D), jnp.float32),
    ),
)

_tc_last = pl.pallas_call(
    _tc_last_body,
    grid=(_B,),
    in_specs=[_blk, _blk, _blk, _blk, _blk, _blk, _wblk, _bblk, _wblk],
    out_specs=[_mblk, _mblk],
    out_shape=(
        jax.ShapeDtypeStruct((_B, 1, _D), jnp.float32),
        jax.ShapeDtypeStruct((_B, 1, _D), jnp.float32),
    ),
)

_tc_cos = pl.pallas_call(
    _tc_cos_body,
    out_shape=jax.ShapeDtypeStruct((_B, 1), jnp.float32),
)


def kernel(src_x, tgt_x, src_edge_attr, tgt_edge_attr, W_rel, b_rel, W_root,
           src_edge_index, tgt_edge_index, src_batch, tgt_batch):
    del src_batch, tgt_batch
    zeros = jnp.zeros((_N, _D), jnp.float32)
    src_s, src_d = src_edge_index[0], src_edge_index[1]
    tgt_s, tgt_d = tgt_edge_index[0], tgt_edge_index[1]
    src_d_r = src_d.reshape(_NTILES, _STEPS, _K)
    tgt_d_r = tgt_d.reshape(_NTILES, _STEPS, _K)

    _sc_counts, _sc_messages = _build_sc_kernels()
    cnt_s, cnt_t = _sc_counts(src_d_r, tgt_d_r, zeros)
    cnt_s = cnt_s.reshape(_B, _NPB, _D)
    cnt_t = cnt_t.reshape(_B, _NPB, _D)

    xs, xt = src_x, tgt_x
    for i in range(_L):
        agg_s, agg_t = _sc_messages(xs, xt, src_s, src_d, src_edge_attr,
                                    tgt_s, tgt_d, tgt_edge_attr, zeros)
        args = (agg_s.reshape(_B, _NPB, _D), agg_t.reshape(_B, _NPB, _D),
                cnt_s, cnt_t,
                xs.reshape(_B, _NPB, _D), xt.reshape(_B, _NPB, _D),
                W_rel[i], b_rel[i].reshape(1, _D), W_root[i])
        if i < _L - 1:
            ys, yt = _tc_layer(*args)
            xs, xt = ys.reshape(_N, _D), yt.reshape(_N, _D)
        else:
            sm, tm = _tc_last(*args)

    return _tc_cos(sm.reshape(_B, _D), tm.reshape(_B, _D)).reshape(_B)


# speedup vs baseline: 2.8315x; 1.0373x over previous
import functools

import jax
import jax.numpy as jnp
from jax import lax
from jax.experimental import pallas as pl
from jax.experimental.pallas import tpu as pltpu
from jax.experimental.pallas import tpu_sc as plsc

_N = 10000
_E = 160000
_D = 128
_B = 16
_NPB = _N // _B
_L = 4

_NTILES = 16
_EP = _E // _NTILES
_K = 40
_STEPS = _EP // _K
_MUL_UNROLL = 8
_CW = 16

_RPT = 624
_TAIL_OFF = _RPT * _NTILES
_TAIL = _N - _TAIL_OFF


def _partitioned_rowcopy(s, copy_fn):
    copy_fn(s * _RPT, _RPT)

    @pl.when(s == _NTILES - 1)
    def _():
        copy_fn(_TAIL_OFF, _TAIL)


def _sc_counts_body(dst_s_hbm, dst_t_hbm, zeros_hbm, out_s, out_t,
                    didx_v, ones_v, cnt_sh, sem_sc):
    c = lax.axis_index("c")
    s = lax.axis_index("s")
    one = jnp.full((16,), 1.0, dtype=jnp.float32)

    def rowinit(i, _):
        ones_v[i, :] = one
        return 0
    lax.fori_loop(0, _K, rowinit, 0)

    _partitioned_rowcopy(s, lambda off, sz: pltpu.sync_copy(
        zeros_hbm.at[pl.ds(off, sz)], cnt_sh.at[pl.ds(off, sz)]))

    def run(dst_hbm, out_hbm, sem):
        pltpu.sync_copy(dst_hbm.at[s], didx_v)
        plsc.subcore_barrier()

        def scat_start(k):
            pltpu.async_copy(ones_v, cnt_sh.at[didx_v.at[k]], sem, add=True)

        def scat_wait(k):
            pltpu.make_async_copy(ones_v, cnt_sh.at[didx_v.at[k]], sem).wait()

        depth = 8
        for k in range(depth):
            scat_start(k)

        def step(k, _):
            scat_wait(k)
            scat_start(k + depth)
            return 0
        lax.fori_loop(0, _STEPS - depth, step, 0)
        for k in range(depth):
            scat_wait(_STEPS - depth + k)
        plsc.subcore_barrier()
        _partitioned_rowcopy(s, lambda off, sz: pltpu.sync_copy(
            cnt_sh.at[pl.ds(off, sz)], out_hbm.at[pl.ds(off, sz)]))

    @pl.when(c == 0)
    def _():
        run(dst_s_hbm, out_s, sem_sc)

    @pl.when(c == 1)
    def _():
        run(dst_t_hbm, out_t, sem_sc)


def _sc_messages_body(xs_hbm, xt_hbm,
                      src_s_hbm, dst_s_hbm, attr_s_hbm,
                      src_t_hbm, dst_t_hbm, attr_t_hbm,
                      zeros_hbm, out_s, out_t,
                      sidx0, sidx1, sidx2, didx0, didx1, didx2,
                      attr0, attr1, attr2, xr0, xr1, xr2, agg_sh,
                      sem_a0, sem_a1, sem_a2, sem_g0, sem_g1, sem_g2,
                      sem_i0, sem_i1, sem_i2, sem_s0, sem_s1, sem_s2):
    c = lax.axis_index("c")
    s = lax.axis_index("s")

    _partitioned_rowcopy(s, lambda off, sz: pltpu.sync_copy(
        zeros_hbm.at[pl.ds(off, sz)], agg_sh.at[pl.ds(off, sz)]))
    plsc.subcore_barrier()

    sidx = (sidx0, sidx1, sidx2)
    didx = (didx0, didx1, didx2)
    attrs = (attr0, attr1, attr2)
    xrs = (xr0, xr1, xr2)
    sem_a = (sem_a0, sem_a1, sem_a2)
    sem_g = (sem_g0, sem_g1, sem_g2)
    sem_i = (sem_i0, sem_i1, sem_i2)
    sem_s = (sem_s0, sem_s1, sem_s2)

    def run(x_hbm, src_hbm, dst_hbm, attr_hbm, out_hbm):
        def idx_copies(k, b):
            base = s * _EP + k * _K
            return (
                pltpu.make_async_copy(src_hbm.at[pl.ds(base, _K)],
                                      sidx[b], sem_i[b]),
                pltpu.make_async_copy(dst_hbm.at[pl.ds(base, _K)],
                                      didx[b], sem_i[b]),
            )

        def attr_copy(k, b):
            base = s * _EP + k * _K
            return pltpu.make_async_copy(attr_hbm.at[pl.ds(base, _K)],
                                         attrs[b], sem_a[b])

        def gather_copy(k, b):
            del k
            return pltpu.make_async_copy(x_hbm.at[sidx[b]], xrs[b], sem_g[b])

        def scat_start(b):
            pltpu.async_copy(attrs[b], agg_sh.at[didx[b]], sem_s[b], add=True)

        def scat_wait(b):
            pltpu.make_async_copy(attrs[b], agg_sh.at[didx[b]],
                                  sem_s[b]).wait()

        def in_start(k, b):
            for cp in idx_copies(k, b):
                cp.start()
            attr_copy(k, b).start()

        def mul(b):
            def mrows(i, _):
                for r in range(_MUL_UNROLL):
                    row = i * _MUL_UNROLL + r
                    for j in range(_D // 16):
                        sl = pl.ds(j * 16, 16)
                        attrs[b][row, sl] = attrs[b][row, sl] * xrs[b][row, sl]
                return 0
            lax.fori_loop(0, _K // _MUL_UNROLL, mrows, 0)

        def step(k, b, b1, b2):
            @pl.when(k + 2 < _STEPS)
            def _():
                scat_wait(b2)
                in_start(k + 2, b2)

            @pl.when(k + 1 < _STEPS)
            def _():
                for cp in idx_copies(k + 1, b1):
                    cp.wait()
                gather_copy(k + 1, b1).start()
            attr_copy(k, b).wait()
            gather_copy(k, b).wait()
            mul(b)
            scat_start(b)

        in_start(0, 0)
        in_start(1, 1)
        for cp in idx_copies(0, 0):
            cp.wait()
        gather_copy(0, 0).start()
        in_start(2, 2)
        for cp in idx_copies(1, 1):
            cp.wait()
        gather_copy(1, 1).start()
        attr_copy(0, 0).wait()
        gather_copy(0, 0).wait()
        mul(0)
        scat_start(0)

        def trip(i, _):
            k0 = 3 * i
            step(k0 + 1, 1, 2, 0)
            step(k0 + 2, 2, 0, 1)
            step(k0 + 3, 0, 1, 2)
            return 0
        lax.fori_loop(0, (_STEPS - 1) // 3, trip, 0)

        scat_wait(1)
        scat_wait(2)
        scat_wait(0)

        plsc.subcore_barrier()
        _partitioned_rowcopy(s, lambda off, sz: pltpu.sync_copy(
            agg_sh.at[pl.ds(off, sz)], out_hbm.at[pl.ds(off, sz)]))

    @pl.when(c == 0)
    def _():
        run(xs_hbm, src_s_hbm, dst_s_hbm, attr_s_hbm, out_s)

    @pl.when(c == 1)
    def _():
        run(xt_hbm, src_t_hbm, dst_t_hbm, attr_t_hbm, out_t)


@functools.lru_cache(maxsize=None)
def _build_sc_kernels():
    mesh = plsc.VectorSubcoreMesh(core_axis_name="c", subcore_axis_name="s",
                                  num_cores=2, num_subcores=_NTILES)
    out2 = (jax.ShapeDtypeStruct((_N, _D), jnp.float32),
            jax.ShapeDtypeStruct((_N, _D), jnp.float32))
    outc = (jax.ShapeDtypeStruct((_N, _CW), jnp.float32),
            jax.ShapeDtypeStruct((_N, _CW), jnp.float32))
    counts = pl.kernel(
        _sc_counts_body, out_type=outc, mesh=mesh,
        scratch_types=[
            pltpu.VMEM((_STEPS, _K), jnp.int32),
            pltpu.VMEM((_K, _CW), jnp.float32),
            pltpu.VMEM_SHARED((_N, _CW), jnp.float32),
            pltpu.SemaphoreType.DMA,
        ])
    messages = pl.kernel(
        _sc_messages_body, out_type=out2, mesh=mesh,
        scratch_types=(
            [pltpu.VMEM((_K,), jnp.int32)] * 6
            + [pltpu.VMEM((_K, _D), jnp.float32)] * 6
            + [pltpu.VMEM_SHARED((_N, _D), jnp.float32)]
            + [pltpu.SemaphoreType.DMA] * 12
        ))
    return counts, messages


def _dense_heads(aggs, aggt, cnts, cntt, xs, xt, Wr, br, Wroot):
    ms = aggs / jnp.maximum(cnts[:, :1], 1.0)
    mt = aggt / jnp.maximum(cntt[:, :1], 1.0)
    hs = jnp.maximum(
        jnp.dot(ms, Wr, preferred_element_type=jnp.float32) + br
        + jnp.dot(xs, Wroot, preferred_element_type=jnp.float32), 0.0)
    ht = jnp.maximum(
        jnp.dot(mt, Wr, preferred_element_type=jnp.float32) + br
        + jnp.dot(xt, Wroot, preferred_element_type=jnp.float32), 0.0)
    return hs, ht


def _attention(hs, ht):
    sim = lax.dot_general(hs, ht, (((1,), (1,)), ((), ())),
                          preferred_element_type=jnp.float32)
    es = jnp.exp(sim - jnp.max(sim, axis=1, keepdims=True))
    attn_s = es / jnp.sum(es, axis=1, keepdims=True)
    new_s = jnp.dot(attn_s, ht, preferred_element_type=jnp.float32)
    et = jnp.exp(sim - jnp.max(sim, axis=0, keepdims=True))
    attn_t = et / jnp.sum(et, axis=0, keepdims=True)
    new_t = lax.dot_general(attn_t, hs, (((0,), (0,)), ((), ())),
                            preferred_element_type=jnp.float32)
    return new_s, new_t


def _tc_layer_body(aggs_ref, aggt_ref, cnts_ref, cntt_ref, xs_ref, xt_ref,
                   Wr_ref, br_ref, Wroot_ref, os_ref, ot_ref):
    hs, ht = _dense_heads(aggs_ref[0], aggt_ref[0], cnts_ref[0], cntt_ref[0],
                          xs_ref[0], xt_ref[0], Wr_ref[...], br_ref[...],
                          Wroot_ref[...])
    new_s, new_t = _attention(hs, ht)
    os_ref[0] = new_s
    ot_ref[0] = new_t


def _tc_last_body(aggs_ref, aggt_ref, cnts_ref, cntt_ref, xs_ref, xt_ref,
                  Wr_ref, br_ref, Wroot_ref, cos_ref):
    hs, ht = _dense_heads(aggs_ref[0], aggt_ref[0], cnts_ref[0], cntt_ref[0],
                          xs_ref[0], xt_ref[0], Wr_ref[...], br_ref[...],
                          Wroot_ref[...])
    new_s, new_t = _attention(hs, ht)
    sm = jnp.mean(new_s, axis=0, keepdims=True)
    tm = jnp.mean(new_t, axis=0, keepdims=True)
    dot = jnp.sum(sm * tm, axis=1, keepdims=True)
    ns = jnp.sqrt(jnp.sum(sm * sm, axis=1, keepdims=True))
    nt = jnp.sqrt(jnp.sum(tm * tm, axis=1, keepdims=True))
    cos_ref[...] = (dot / jnp.maximum(ns * nt, 1e-8))[None]


_blk = pl.BlockSpec((1, _NPB, _D), lambda b: (b, 0, 0))
_cblk = pl.BlockSpec((1, _NPB, _CW), lambda b: (b, 0, 0))
_wblk = pl.BlockSpec((_D, _D), lambda b: (0, 0))
_bblk = pl.BlockSpec((1, _D), lambda b: (0, 0))
_mblk = pl.BlockSpec((1, 1, 1), lambda b: (b, 0, 0))

_tc_layer = pl.pallas_call(
    _tc_layer_body,
    grid=(_B,),
    in_specs=[_blk, _blk, _cblk, _cblk, _blk, _blk, _wblk, _bblk, _wblk],
    out_specs=[_blk, _blk],
    out_shape=(
        jax.ShapeDtypeStruct((_B, _NPB, _D), jnp.float32),
        jax.ShapeDtypeStruct((_B, _NPB, _D), jnp.float32),
    ),
)

_tc_last = pl.pallas_call(
    _tc_last_body,
    grid=(_B,),
    in_specs=[_blk, _blk, _cblk, _cblk, _blk, _blk, _wblk, _bblk, _wblk],
    out_specs=[_mblk],
    out_shape=(jax.ShapeDtypeStruct((_B, 1, 1), jnp.float32),),
)


def kernel(src_x, tgt_x, src_edge_attr, tgt_edge_attr, W_rel, b_rel, W_root,
           src_edge_index, tgt_edge_index, src_batch, tgt_batch):
    del src_batch, tgt_batch
    zeros = jnp.zeros((_N, _D), jnp.float32)
    src_s, src_d = src_edge_index[0], src_edge_index[1]
    tgt_s, tgt_d = tgt_edge_index[0], tgt_edge_index[1]
    src_d_r = src_d.reshape(_NTILES, _STEPS, _K)
    tgt_d_r = tgt_d.reshape(_NTILES, _STEPS, _K)

    _sc_counts, _sc_messages = _build_sc_kernels()
    cnt_s, cnt_t = _sc_counts(src_d_r, tgt_d_r, jnp.zeros((_N, _CW), jnp.float32))
    cnt_s = cnt_s.reshape(_B, _NPB, _CW)
    cnt_t = cnt_t.reshape(_B, _NPB, _CW)

    xs, xt = src_x, tgt_x
    for i in range(_L):
        agg_s, agg_t = _sc_messages(xs, xt, src_s, src_d, src_edge_attr,
                                    tgt_s, tgt_d, tgt_edge_attr, zeros)
        args = (agg_s.reshape(_B, _NPB, _D), agg_t.reshape(_B, _NPB, _D),
                cnt_s, cnt_t,
                xs.reshape(_B, _NPB, _D), xt.reshape(_B, _NPB, _D),
                W_rel[i], b_rel[i].reshape(1, _D), W_root[i])
        if i < _L - 1:
            ys, yt = _tc_layer(*args)
            xs, xt = ys.reshape(_N, _D), yt.reshape(_N, _D)
        else:
            cos = _tc_last(*args)[0]

    return cos.reshape(_B)
